# baseline (speedup 1.0000x reference)
import functools

import jax
import jax.numpy as jnp
from jax.experimental import pallas as pl
from jax.experimental.pallas import tpu as pltpu

SEQ_LEN = 4096
BSZ = 2
EMBED_DIM = 4096
N_EXPERTS = 31
TOP_K = 3
ALPHA = 0.001

E_PAD = 128
ROW_BLOCK = 256
N_ROWS = SEQ_LEN * BSZ
N_BLOCKS = N_ROWS // ROW_BLOCK
NEG = -1e30


def _gate_kernel(x_ref, w_ref, idx_ref, wgt_ref, aux_ref, cnt_acc, sum_acc):
    i = pl.program_id(0)

    @pl.when(i == 0)
    def _init():
        cnt_acc[...] = jnp.zeros_like(cnt_acc)
        sum_acc[...] = jnp.zeros_like(sum_acc)

    logits = jax.lax.dot_general(
        x_ref[...], w_ref[...],
        dimension_numbers=(((1,), (0,)), ((), ())),
        preferred_element_type=jnp.float32,
        precision=jax.lax.Precision.DEFAULT,
    )
    lane = jax.lax.broadcasted_iota(jnp.int32, (ROW_BLOCK, E_PAD), 1)
    logits = jnp.where(lane < N_EXPERTS, logits, NEG)

    m = jnp.max(logits, axis=-1, keepdims=True)
    p = jnp.exp(logits - m)
    scores = p / jnp.sum(p, axis=-1, keepdims=True)

    cur = scores
    vals = []
    idxs = []
    for _ in range(TOP_K):
        v = jnp.max(cur, axis=-1, keepdims=True)
        hit = cur >= v
        ix = jnp.min(jnp.where(hit, lane, E_PAD), axis=-1, keepdims=True)
        vals.append(v)
        idxs.append(ix)
        cur = jnp.where(lane == ix, -1.0, cur)

    topv = jnp.concatenate(vals, axis=-1)
    topi = jnp.concatenate(idxs, axis=-1)
    denom = jnp.sum(topv, axis=-1, keepdims=True) + 1e-20
    idx_ref[...] = topi
    wgt_ref[...] = topv / denom

    h = (i * ROW_BLOCK) // SEQ_LEN
    onehot_h = (jax.lax.broadcasted_iota(jnp.int32, (2, 1), 0) == h).astype(jnp.float32)

    sum_acc[...] += onehot_h * jnp.sum(scores, axis=0)[None, :]
    cnt = jnp.zeros((E_PAD,), dtype=jnp.float32)
    for j in range(TOP_K):
        cnt += jnp.sum((lane == topi[:, j:j + 1]).astype(jnp.float32), axis=0)
    cnt_acc[...] += onehot_h * cnt[None, :]

    @pl.when(i == N_BLOCKS - 1)
    def _finalize():
        scale = ALPHA * (1.0 / BSZ) * N_EXPERTS / (SEQ_LEN * SEQ_LEN * TOP_K)
        aux_ref[...] = (jnp.sum(cnt_acc[...] * sum_acc[...]) * scale).reshape(1, 1)


@functools.partial(jax.jit, static_argnums=())
def _gate(x_flat, w_pad):
    idx, wgt, aux = pl.pallas_call(
        _gate_kernel,
        grid=(N_BLOCKS,),
        in_specs=[
            pl.BlockSpec((ROW_BLOCK, EMBED_DIM), lambda i: (i, 0)),
            pl.BlockSpec((EMBED_DIM, E_PAD), lambda i: (0, 0)),
        ],
        out_specs=[
            pl.BlockSpec((ROW_BLOCK, TOP_K), lambda i: (i, 0)),
            pl.BlockSpec((ROW_BLOCK, TOP_K), lambda i: (i, 0)),
            pl.BlockSpec((1, 1), lambda i: (0, 0)),
        ],
        out_shape=[
            jax.ShapeDtypeStruct((N_ROWS, TOP_K), jnp.int32),
            jax.ShapeDtypeStruct((N_ROWS, TOP_K), jnp.float32),
            jax.ShapeDtypeStruct((1, 1), jnp.float32),
        ],
        scratch_shapes=[
            pltpu.VMEM((2, E_PAD), jnp.float32),
            pltpu.VMEM((2, E_PAD), jnp.float32),
        ],
    )(x_flat, w_pad)
    return idx, wgt, aux[0, 0]


def kernel(x, weight):
    x_flat = x.reshape(-1, EMBED_DIM)
    w_pad = jnp.zeros((EMBED_DIM, E_PAD), dtype=weight.dtype).at[:, :N_EXPERTS].set(weight.T)
    return _gate(x_flat, w_pad)


# speedup vs baseline: 1.0417x; 1.0417x over previous
import functools

import jax
import jax.numpy as jnp
from jax.experimental import pallas as pl
from jax.experimental.pallas import tpu as pltpu

SEQ_LEN = 4096
BSZ = 2
EMBED_DIM = 4096
N_EXPERTS = 31
TOP_K = 3
ALPHA = 0.001

E_PAD = 128
ROW_BLOCK = 512
N_ROWS = SEQ_LEN * BSZ
N_BLOCKS = N_ROWS // ROW_BLOCK
NEG = -1e30


def _gate_kernel(x_ref, w_ref, idx_ref, wgt_ref, aux_ref, cnt_acc, sum_acc):
    i = pl.program_id(0)

    @pl.when(i == 0)
    def _init():
        cnt_acc[...] = jnp.zeros_like(cnt_acc)
        sum_acc[...] = jnp.zeros_like(sum_acc)

    logits = jax.lax.dot_general(
        x_ref[...], w_ref[...],
        dimension_numbers=(((1,), (0,)), ((), ())),
        preferred_element_type=jnp.float32,
        precision=jax.lax.Precision.DEFAULT,
    )
    lane = jax.lax.broadcasted_iota(jnp.int32, (ROW_BLOCK, E_PAD), 1)
    logits = jnp.where(lane < N_EXPERTS, logits, NEG)

    m = jnp.max(logits, axis=-1, keepdims=True)
    p = jnp.exp(logits - m)
    scores = p / jnp.sum(p, axis=-1, keepdims=True)

    cur = scores
    vals = []
    idxs = []
    for _ in range(TOP_K):
        v = jnp.max(cur, axis=-1, keepdims=True)
        hit = cur >= v
        ix = jnp.min(jnp.where(hit, lane, E_PAD), axis=-1, keepdims=True)
        vals.append(v)
        idxs.append(ix)
        cur = jnp.where(lane == ix, -1.0, cur)

    topv = jnp.concatenate(vals, axis=-1)
    topi = jnp.concatenate(idxs, axis=-1)
    denom = jnp.sum(topv, axis=-1, keepdims=True) + 1e-20
    idx_ref[...] = topi
    wgt_ref[...] = topv / denom

    h = (i * ROW_BLOCK) // SEQ_LEN
    onehot_h = (jax.lax.broadcasted_iota(jnp.int32, (2, 1), 0) == h).astype(jnp.float32)

    sum_acc[...] += onehot_h * jnp.sum(scores, axis=0)[None, :]
    cnt = jnp.zeros((E_PAD,), dtype=jnp.float32)
    for j in range(TOP_K):
        cnt += jnp.sum((lane == topi[:, j:j + 1]).astype(jnp.float32), axis=0)
    cnt_acc[...] += onehot_h * cnt[None, :]

    @pl.when(i == N_BLOCKS - 1)
    def _finalize():
        scale = ALPHA * (1.0 / BSZ) * N_EXPERTS / (SEQ_LEN * SEQ_LEN * TOP_K)
        aux_ref[...] = (jnp.sum(cnt_acc[...] * sum_acc[...]) * scale).reshape(1, 1)


@functools.partial(jax.jit, static_argnums=())
def _gate(x_flat, w_pad):
    idx, wgt, aux = pl.pallas_call(
        _gate_kernel,
        grid=(N_BLOCKS,),
        in_specs=[
            pl.BlockSpec((ROW_BLOCK, EMBED_DIM), lambda i: (i, 0)),
            pl.BlockSpec((EMBED_DIM, E_PAD), lambda i: (0, 0)),
        ],
        out_specs=[
            pl.BlockSpec((ROW_BLOCK, TOP_K), lambda i: (i, 0)),
            pl.BlockSpec((ROW_BLOCK, TOP_K), lambda i: (i, 0)),
            pl.BlockSpec((1, 1), lambda i: (0, 0)),
        ],
        out_shape=[
            jax.ShapeDtypeStruct((N_ROWS, TOP_K), jnp.int32),
            jax.ShapeDtypeStruct((N_ROWS, TOP_K), jnp.float32),
            jax.ShapeDtypeStruct((1, 1), jnp.float32),
        ],
        scratch_shapes=[
            pltpu.VMEM((2, E_PAD), jnp.float32),
            pltpu.VMEM((2, E_PAD), jnp.float32),
        ],
    )(x_flat, w_pad)
    return idx, wgt, aux[0, 0]


def kernel(x, weight):
    x_flat = x.reshape(-1, EMBED_DIM)
    w_pad = jnp.zeros((EMBED_DIM, E_PAD), dtype=weight.dtype).at[:, :N_EXPERTS].set(weight.T)
    return _gate(x_flat, w_pad)
